# Initial kernel scaffold; baseline (speedup 1.0000x reference)
import jax
import jax.numpy as jnp
from jax.experimental import pallas as pl


def kernel(logits, mask, targets):
    raise NotImplementedError("write your pallas kernel here")


# speedup vs baseline: 3.0266x; 3.0266x over previous
import jax
import jax.numpy as jnp
from jax.experimental import pallas as pl
from jax.experimental.pallas import tpu as pltpu

_THRESH = (0.5333333611488342, 0.6000000238418579, 0.6666666865348816,
           0.7333333492279053, 0.8000000715255737, 0.8666667342185974,
           0.9333333969116211)

_ROWS = 8192
_COLS = 2048
_BLOCK_ROWS = 256
_GRID = _ROWS // _BLOCK_ROWS


def _ece_body(x_ref, m_ref, t_ref, out_ref):
    @pl.when(pl.program_id(0) == 0)
    def _init():
        for k in range(24):
            out_ref[k] = 0.0

    x = x_ref[...]
    mf = m_ref[...].astype(jnp.float32)
    t = t_ref[...]
    conf = 0.5 * jnp.tanh(0.5 * jnp.abs(x)) + 0.5
    acc = jnp.where(x > 0, t, 1.0 - t) * mf
    confm = conf * mf
    out_ref[0] += jnp.sum(mf)
    out_ref[1] += jnp.sum(confm)
    out_ref[2] += jnp.sum(acc)
    for k, th in enumerate(_THRESH):
        g = conf > th
        base = 3 * (k + 1)
        out_ref[base + 0] += jnp.sum(jnp.where(g, mf, 0.0))
        out_ref[base + 1] += jnp.sum(jnp.where(g, confm, 0.0))
        out_ref[base + 2] += jnp.sum(jnp.where(g, acc, 0.0))


def _partial_sums(logits, mask, targets, interpret=False):
    blk = pl.BlockSpec((_BLOCK_ROWS, _COLS), lambda i: (i, 0))
    return pl.pallas_call(
        _ece_body,
        grid=(_GRID,),
        in_specs=[blk, blk, blk],
        out_specs=pl.BlockSpec(memory_space=pltpu.SMEM),
        out_shape=jax.ShapeDtypeStruct((24,), jnp.float32),
        interpret=interpret,
    )(logits, mask, targets)


def kernel(logits, mask, targets):
    part = _partial_sums(logits, mask, targets)
    cum = part.reshape(8, 3)
    zero = jnp.zeros((1, 3), jnp.float32)
    per_bin = cum - jnp.concatenate([cum[1:], zero], axis=0)
    count = per_bin[:, 0]
    sum_conf = per_bin[:, 1]
    sum_acc = per_bin[:, 2]
    total = jnp.float32(logits.size)
    denom = jnp.maximum(count, 1.0)
    contrib = jnp.where(
        count > 0.0,
        jnp.abs(sum_conf / denom - sum_acc / denom) * (count / total),
        0.0,
    )
    return jnp.sum(contrib, keepdims=True)
